# baseline (speedup 1.0000x reference)
import functools

import jax
import jax.numpy as jnp
from jax import lax
from jax.experimental import pallas as pl
from jax.experimental.pallas import tpu as pltpu
from jax.experimental.pallas import tpu_sc as plsc

_N = 10000
_IN = 128
_HID = 128
_OUT = 64
_E1 = 320000
_E2 = 640000

_NC = 2
_NS = 16
_NW = _NC * _NS
_K = 128
_C1 = -(-_E1 // (_NW * _K))
_C2 = -(-_E2 // (_NW * _K))
_E1P = _NW * _K * _C1
_E2P = _NW * _K * _C2
_RPT = 624
_TAIL = _N - _NS * _RPT


def _dense_body(x_ref, w1_ref, wout_ref, b_ref, dense_ref, t1_ref, t2_ref):
    dims = (((1,), (1,)), ((), ()))
    h0 = lax.dot_general(x_ref[...], w1_ref[...], dims,
                         preferred_element_type=jnp.float32)
    wa = wout_ref[:, 0:_HID]
    wb = wout_ref[:, _HID:2 * _HID]
    wc = wout_ref[:, 2 * _HID:3 * _HID]
    dense_ref[...] = lax.dot_general(h0, wa, dims,
                                     preferred_element_type=jnp.float32) + b_ref[0:1, :]
    t1_ref[...] = lax.dot_general(h0, wb, dims, preferred_element_type=jnp.float32)
    t2_ref[...] = lax.dot_general(h0, wc, dims, preferred_element_type=jnp.float32)


def _combine_body(dense_ref, p_ref, out_ref):
    out_ref[...] = dense_ref[...] + p_ref[0] + p_ref[1]


def _sc_body(src1h, dst1h, val1h, src2h, dst2h, val2h, t1h, t2h, zh, outh,
             acc, sb, db, vb, rows_a, rows_b, sem_ga, sem_gb):
    c = lax.axis_index("c")
    s = lax.axis_index("s")
    wid = c * _NS + s

    pltpu.sync_copy(zh.at[pl.ds(s * _RPT, _RPT)], acc.at[pl.ds(s * _RPT, _RPT)])

    @pl.when(s == 0)
    def _():
        pltpu.sync_copy(zh.at[pl.ds(_NS * _RPT, _TAIL)],
                        acc.at[pl.ds(_NS * _RPT, _TAIL)])

    plsc.subcore_barrier()

    def mult(rows, k):
        for g in range(_K // 16):
            vv = vb[k, pl.ds(g * 16, 16)]
            for jj in range(16):
                j = g * 16 + jj
                v = vv[jj]
                for q in range(_OUT // 16):
                    sl = pl.ds(q * 16, 16)
                    rows[j, sl] = rows[j, sl] * v

    def run_list(nch, srch, dsth, valh, th):
        pltpu.sync_copy(srch.at[wid], sb.at[pl.ds(0, nch)])
        pltpu.sync_copy(dsth.at[wid], db.at[pl.ds(0, nch)])
        pltpu.sync_copy(valh.at[wid], vb.at[pl.ds(0, nch)])

        base = nch % 2
        if base:
            pltpu.async_copy(th.at[sb.at[0]], rows_a, sem_ga).wait()
            mult(rows_a, 0)
            pltpu.sync_copy(rows_a, acc.at[db.at[0]], add=True)
        pltpu.async_copy(th.at[sb.at[base]], rows_a, sem_ga)

        def body(i, carry):
            a = base + 2 * i
            b = a + 1
            nxt = jnp.minimum(a + 2, nch - 1)
            pltpu.make_async_copy(th.at[sb.at[a]], rows_a, sem_ga).wait()
            pltpu.async_copy(th.at[sb.at[b]], rows_b, sem_gb)
            mult(rows_a, a)
            pltpu.sync_copy(rows_a, acc.at[db.at[a]], add=True)
            pltpu.make_async_copy(th.at[sb.at[b]], rows_b, sem_gb).wait()
            pltpu.async_copy(th.at[sb.at[nxt]], rows_a, sem_ga)
            mult(rows_b, b)
            pltpu.sync_copy(rows_b, acc.at[db.at[b]], add=True)
            return carry
        lax.fori_loop(0, (nch - base) // 2, body, 0)
        pltpu.make_async_copy(th.at[sb.at[0]], rows_a, sem_ga).wait()

    run_list(_C1, src1h, dst1h, val1h, t1h)
    run_list(_C2, src2h, dst2h, val2h, t2h)
    plsc.subcore_barrier()

    pltpu.sync_copy(acc.at[pl.ds(s * _RPT, _RPT)],
                    outh.at[c, pl.ds(s * _RPT, _RPT)])

    @pl.when(s == 0)
    def _():
        pltpu.sync_copy(acc.at[pl.ds(_NS * _RPT, _TAIL)],
                        outh.at[c, pl.ds(_NS * _RPT, _TAIL)])


def kernel(x, edge_index, adj_values, adj2_index, adj2_values, W1, W_out, b_out):
    f32 = jnp.float32

    rblk = 2000
    grid = (_N // rblk,)
    dense, t1, t2 = pl.pallas_call(
        _dense_body,
        grid=grid,
        in_specs=[
            pl.BlockSpec((rblk, _IN), lambda i: (i, 0)),
            pl.BlockSpec((_HID, _IN), lambda i: (0, 0)),
            pl.BlockSpec((_OUT, 3 * _HID), lambda i: (0, 0)),
            pl.BlockSpec((8, _OUT), lambda i: (0, 0)),
        ],
        out_specs=[
            pl.BlockSpec((rblk, _OUT), lambda i: (i, 0)),
            pl.BlockSpec((rblk, _OUT), lambda i: (i, 0)),
            pl.BlockSpec((rblk, _OUT), lambda i: (i, 0)),
        ],
        out_shape=[
            jax.ShapeDtypeStruct((_N, _OUT), f32),
            jax.ShapeDtypeStruct((_N, _OUT), f32),
            jax.ShapeDtypeStruct((_N, _OUT), f32),
        ],
    )(x, W1, W_out, jnp.broadcast_to(b_out, (8, _OUT)))

    p1 = [(0, _E1P - _E1)]
    p2 = [(0, _E2P - _E2)]
    dst1 = jnp.pad(edge_index[0], p1).reshape(_NW, _C1, _K)
    src1 = jnp.pad(edge_index[1], p1).reshape(_NW, _C1, _K)
    val1 = jnp.pad(adj_values, p1).reshape(_NW, _C1, _K)
    dst2 = jnp.pad(adj2_index[0], p2).reshape(_NW, _C2, _K)
    src2 = jnp.pad(adj2_index[1], p2).reshape(_NW, _C2, _K)
    val2 = jnp.pad(adj2_values, p2).reshape(_NW, _C2, _K)
    zeros = jnp.zeros((_N, _OUT), f32)

    mesh = plsc.VectorSubcoreMesh(core_axis_name="c", subcore_axis_name="s",
                                  num_cores=_NC, num_subcores=_NS)
    partial = pl.kernel(
        _sc_body,
        jax.ShapeDtypeStruct((_NC, _N, _OUT), f32),
        mesh=mesh,
        compiler_params=pltpu.CompilerParams(use_tc_tiling_on_sc=False),
        scratch_types=[
            pltpu.VMEM_SHARED((_N, _OUT), f32),
            pltpu.VMEM((_C2, _K), jnp.int32),
            pltpu.VMEM((_C2, _K), jnp.int32),
            pltpu.VMEM((_C2, _K), f32),
            pltpu.VMEM((_K, _OUT), f32),
            pltpu.VMEM((_K, _OUT), f32),
            pltpu.SemaphoreType.DMA,
            pltpu.SemaphoreType.DMA,
        ],
    )(src1, dst1, val1, src2, dst2, val2, t1, t2, zeros)

    out = pl.pallas_call(
        _combine_body,
        grid=grid,
        in_specs=[
            pl.BlockSpec((rblk, _OUT), lambda i: (i, 0)),
            pl.BlockSpec((_NC, rblk, _OUT), lambda i: (0, i, 0)),
        ],
        out_specs=pl.BlockSpec((rblk, _OUT), lambda i: (i, 0)),
        out_shape=jax.ShapeDtypeStruct((_N, _OUT), f32),
    )(dense, partial)
    return out


# speedup vs baseline: 1.2604x; 1.2604x over previous
import functools

import jax
import jax.numpy as jnp
from jax import lax
from jax.experimental import pallas as pl
from jax.experimental.pallas import tpu as pltpu
from jax.experimental.pallas import tpu_sc as plsc

_N = 10000
_IN = 128
_HID = 128
_OUT = 64
_E1 = 320000
_E2 = 640000

_NC = 2
_NS = 16
_NW = _NC * _NS
_K = 80
_C1 = -(-_E1 // (_NW * _K))
_C2 = -(-_E2 // (_NW * _K))
_E1P = _NW * _K * _C1
_E2P = _NW * _K * _C2
_RPT = 624
_TAIL = _N - _NS * _RPT


def _dense_body(x_ref, w1_ref, wout_ref, b_ref, dense_ref, t1_ref, t2_ref):
    dims = (((1,), (1,)), ((), ()))
    h0 = lax.dot_general(x_ref[...], w1_ref[...], dims,
                         preferred_element_type=jnp.float32)
    wa = wout_ref[:, 0:_HID]
    wb = wout_ref[:, _HID:2 * _HID]
    wc = wout_ref[:, 2 * _HID:3 * _HID]
    dense_ref[...] = lax.dot_general(h0, wa, dims,
                                     preferred_element_type=jnp.float32) + b_ref[0:1, :]
    t1_ref[...] = lax.dot_general(h0, wb, dims, preferred_element_type=jnp.float32)
    t2_ref[...] = lax.dot_general(h0, wc, dims, preferred_element_type=jnp.float32)


def _combine_body(dense_ref, p_ref, out_ref):
    out_ref[...] = dense_ref[...] + p_ref[0] + p_ref[1]


def _sc_body(src1h, dst1h, val1h, src2h, dst2h, val2h, t1h, t2h, zh, outh,
             acc, tab2, sb, db, vb, rows_a, rows_b, sem_ga, sem_gb, sem_t):
    c = lax.axis_index("c")
    s = lax.axis_index("s")
    wid = c * _NS + s

    tcp = pltpu.make_async_copy(t2h.at[pl.ds(s * _RPT, _RPT)],
                                tab2.at[pl.ds(s * _RPT, _RPT)], sem_t)
    tcp.start()

    @pl.when(s == 0)
    def _():
        pltpu.async_copy(t2h.at[pl.ds(_NS * _RPT, _TAIL)],
                         tab2.at[pl.ds(_NS * _RPT, _TAIL)], sem_t)

    pltpu.sync_copy(zh.at[pl.ds(s * _RPT, _RPT)], acc.at[pl.ds(s * _RPT, _RPT)])

    @pl.when(s == 0)
    def _():
        pltpu.sync_copy(zh.at[pl.ds(_NS * _RPT, _TAIL)],
                        acc.at[pl.ds(_NS * _RPT, _TAIL)])

    plsc.subcore_barrier()

    def mult(rows, k):
        for g in range(_K // 16):
            vv = vb[k, pl.ds(g * 16, 16)]
            for jj in range(16):
                j = g * 16 + jj
                v = vv[jj]
                for q in range(_OUT // 16):
                    sl = pl.ds(q * 16, 16)
                    rows[j, sl] = rows[j, sl] * v

    def run_list(nch, srcsl, dstsl, valsl, th):
        pltpu.sync_copy(srcsl, sb.at[pl.ds(0, nch)])
        pltpu.sync_copy(dstsl, db.at[pl.ds(0, nch)])
        pltpu.sync_copy(valsl, vb.at[pl.ds(0, nch)])

        base = nch % 2
        if base:
            pltpu.async_copy(th.at[sb.at[0]], rows_a, sem_ga).wait()
            mult(rows_a, 0)
            pltpu.sync_copy(rows_a, acc.at[db.at[0]], add=True)
        pltpu.async_copy(th.at[sb.at[base]], rows_a, sem_ga)

        def body(i, carry):
            a = base + 2 * i
            b = a + 1
            nxt = jnp.minimum(a + 2, nch - 1)
            pltpu.make_async_copy(th.at[sb.at[a]], rows_a, sem_ga).wait()
            pltpu.async_copy(th.at[sb.at[b]], rows_b, sem_gb)
            mult(rows_a, a)
            pltpu.sync_copy(rows_a, acc.at[db.at[a]], add=True)
            pltpu.make_async_copy(th.at[sb.at[b]], rows_b, sem_gb).wait()
            pltpu.async_copy(th.at[sb.at[nxt]], rows_a, sem_ga)
            mult(rows_b, b)
            pltpu.sync_copy(rows_b, acc.at[db.at[b]], add=True)
            return carry
        lax.fori_loop(0, (nch - base) // 2, body, 0)
        pltpu.make_async_copy(th.at[sb.at[0]], rows_a, sem_ga).wait()

    run_list(_C1, src1h.at[wid], dst1h.at[wid], val1h.at[wid], t1h)

    tcp.wait()

    @pl.when(s == 0)
    def _():
        pltpu.make_async_copy(t2h.at[pl.ds(_NS * _RPT, _TAIL)],
                              tab2.at[pl.ds(_NS * _RPT, _TAIL)], sem_t).wait()

    plsc.subcore_barrier()

    run_list(_C1, src2h.at[wid, 0], dst2h.at[wid, 0], val2h.at[wid, 0], tab2)
    run_list(_C1, src2h.at[wid, 1], dst2h.at[wid, 1], val2h.at[wid, 1], tab2)
    plsc.subcore_barrier()

    pltpu.sync_copy(acc.at[pl.ds(s * _RPT, _RPT)],
                    outh.at[c, pl.ds(s * _RPT, _RPT)])

    @pl.when(s == 0)
    def _():
        pltpu.sync_copy(acc.at[pl.ds(_NS * _RPT, _TAIL)],
                        outh.at[c, pl.ds(_NS * _RPT, _TAIL)])


def kernel(x, edge_index, adj_values, adj2_index, adj2_values, W1, W_out, b_out):
    f32 = jnp.float32

    rblk = 2000
    grid = (_N // rblk,)
    dense, t1, t2 = pl.pallas_call(
        _dense_body,
        grid=grid,
        in_specs=[
            pl.BlockSpec((rblk, _IN), lambda i: (i, 0)),
            pl.BlockSpec((_HID, _IN), lambda i: (0, 0)),
            pl.BlockSpec((_OUT, 3 * _HID), lambda i: (0, 0)),
            pl.BlockSpec((8, _OUT), lambda i: (0, 0)),
        ],
        out_specs=[
            pl.BlockSpec((rblk, _OUT), lambda i: (i, 0)),
            pl.BlockSpec((rblk, _OUT), lambda i: (i, 0)),
            pl.BlockSpec((rblk, _OUT), lambda i: (i, 0)),
        ],
        out_shape=[
            jax.ShapeDtypeStruct((_N, _OUT), f32),
            jax.ShapeDtypeStruct((_N, _OUT), f32),
            jax.ShapeDtypeStruct((_N, _OUT), f32),
        ],
    )(x, W1, W_out, jnp.broadcast_to(b_out, (8, _OUT)))

    p1 = [(0, _E1P - _E1)]
    p2 = [(0, _E2P - _E2)]
    dst1 = jnp.pad(edge_index[0], p1).reshape(_NW, _C1, _K)
    src1 = jnp.pad(edge_index[1], p1).reshape(_NW, _C1, _K)
    val1 = jnp.pad(adj_values, p1).reshape(_NW, _C1, _K)
    dst2 = jnp.pad(adj2_index[0], p2).reshape(_NW, 2, _C1, _K)
    src2 = jnp.pad(adj2_index[1], p2).reshape(_NW, 2, _C1, _K)
    val2 = jnp.pad(adj2_values, p2).reshape(_NW, 2, _C1, _K)
    zeros = jnp.zeros((_N, _OUT), f32)

    mesh = plsc.VectorSubcoreMesh(core_axis_name="c", subcore_axis_name="s",
                                  num_cores=_NC, num_subcores=_NS)
    partial = pl.kernel(
        _sc_body,
        jax.ShapeDtypeStruct((_NC, _N, _OUT), f32),
        mesh=mesh,
        compiler_params=pltpu.CompilerParams(use_tc_tiling_on_sc=False),
        scratch_types=[
            pltpu.VMEM_SHARED((_N, _OUT), f32),
            pltpu.VMEM_SHARED((_N, _OUT), f32),
            pltpu.VMEM((_C1, _K), jnp.int32),
            pltpu.VMEM((_C1, _K), jnp.int32),
            pltpu.VMEM((_C1, _K), f32),
            pltpu.VMEM((_K, _OUT), f32),
            pltpu.VMEM((_K, _OUT), f32),
            pltpu.SemaphoreType.DMA,
            pltpu.SemaphoreType.DMA,
            pltpu.SemaphoreType.DMA,
        ],
    )(src1, dst1, val1, src2, dst2, val2, t1, t2, zeros)

    out = pl.pallas_call(
        _combine_body,
        grid=grid,
        in_specs=[
            pl.BlockSpec((rblk, _OUT), lambda i: (i, 0)),
            pl.BlockSpec((_NC, rblk, _OUT), lambda i: (0, i, 0)),
        ],
        out_specs=pl.BlockSpec((rblk, _OUT), lambda i: (i, 0)),
        out_shape=jax.ShapeDtypeStruct((_N, _OUT), f32),
    )(dense, partial)
    return out


# speedup vs baseline: 1.3966x; 1.1081x over previous
import functools

import jax
import jax.numpy as jnp
import numpy as np
from jax import lax
from jax.experimental import pallas as pl
from jax.experimental.pallas import tpu as pltpu
from jax.experimental.pallas import tpu_sc as plsc

_N = 10000
_IN = 128
_HID = 128
_OUT = 64
_E1 = 320000
_E2 = 640000

_NC = 2
_NS = 16
_NW = _NC * _NS
_K = 80
_C1 = -(-_E1 // (_NW * _K))
_C2 = -(-_E2 // (_NW * _K))
_E1P = _NW * _K * _C1
_E2P = _NW * _K * _C2
_RPT = 624
_TAIL = _N - _NS * _RPT

_PERM = np.empty(_OUT, np.int32)
for _g in range(0, _OUT, 32):
    for _t in range(16):
        _PERM[_g + 2 * _t] = _g + _t
        _PERM[_g + 2 * _t + 1] = _g + 16 + _t


def _dense_body(x_ref, w1_ref, wa_ref, wbp_ref, wcp_ref, b_ref,
                dense_ref, t1_ref, t2_ref):
    dims = (((1,), (1,)), ((), ()))
    h0 = lax.dot_general(x_ref[...], w1_ref[...], dims,
                         preferred_element_type=jnp.float32)
    dense_ref[...] = lax.dot_general(h0, wa_ref[...], dims,
                                     preferred_element_type=jnp.float32) + b_ref[0:1, :]
    t1_ref[...] = lax.dot_general(h0, wbp_ref[...], dims,
                                  preferred_element_type=jnp.float32).astype(jnp.bfloat16)
    t2_ref[...] = lax.dot_general(h0, wcp_ref[...], dims,
                                  preferred_element_type=jnp.float32).astype(jnp.bfloat16)


def _combine_body(dense_ref, p_ref, out_ref):
    out_ref[...] = dense_ref[...] + p_ref[0] + p_ref[1]


def _sc_body(src1h, dst1h, val1h, src2h, dst2h, val2h, t1h, t2h, zh, outh,
             acc, tab1, tab2, sb, db, vb, gb_a, gb_b, fbuf,
             sem_ga, sem_gb, sem_t):
    c = lax.axis_index("c")
    s = lax.axis_index("s")
    wid = c * _NS + s

    t1cp = pltpu.make_async_copy(t1h.at[pl.ds(s * _RPT, _RPT)],
                                 tab1.at[pl.ds(s * _RPT, _RPT)], sem_t)
    t1cp.start()
    t2cp = pltpu.make_async_copy(t2h.at[pl.ds(s * _RPT, _RPT)],
                                 tab2.at[pl.ds(s * _RPT, _RPT)], sem_t)
    t2cp.start()

    @pl.when(s == 0)
    def _():
        pltpu.async_copy(t1h.at[pl.ds(_NS * _RPT, _TAIL)],
                         tab1.at[pl.ds(_NS * _RPT, _TAIL)], sem_t)
        pltpu.async_copy(t2h.at[pl.ds(_NS * _RPT, _TAIL)],
                         tab2.at[pl.ds(_NS * _RPT, _TAIL)], sem_t)

    pltpu.sync_copy(zh.at[pl.ds(s * _RPT, _RPT)], acc.at[pl.ds(s * _RPT, _RPT)])

    @pl.when(s == 0)
    def _():
        pltpu.sync_copy(zh.at[pl.ds(_NS * _RPT, _TAIL)],
                        acc.at[pl.ds(_NS * _RPT, _TAIL)])

    t1cp.wait()
    t2cp.wait()

    @pl.when(s == 0)
    def _():
        pltpu.make_async_copy(t1h.at[pl.ds(_NS * _RPT, _TAIL)],
                              tab1.at[pl.ds(_NS * _RPT, _TAIL)], sem_t).wait()
        pltpu.make_async_copy(t2h.at[pl.ds(_NS * _RPT, _TAIL)],
                              tab2.at[pl.ds(_NS * _RPT, _TAIL)], sem_t).wait()

    plsc.subcore_barrier()

    def mult(gb, k):
        for g in range(_K // 16):
            vv = vb[k, pl.ds(g * 16, 16)]
            for jj in range(16):
                j = g * 16 + jj
                v = vv[jj]
                for h in range(_OUT // 32):
                    x32 = gb[j, pl.ds(h * 32, 32)]
                    lo, hi = plsc.unpack(x32, format=plsc.PackFormat.INTERLEAVED)
                    fbuf[j, pl.ds(h * 32, 16)] = lo * v
                    fbuf[j, pl.ds(h * 32 + 16, 16)] = hi * v

    def run_list(nch, srcsl, dstsl, valsl, tab):
        pltpu.sync_copy(srcsl, sb.at[pl.ds(0, nch)])
        pltpu.sync_copy(dstsl, db.at[pl.ds(0, nch)])
        pltpu.sync_copy(valsl, vb.at[pl.ds(0, nch)])

        base = nch % 2
        if base:
            pltpu.async_copy(tab.at[sb.at[0]], gb_a, sem_ga).wait()
            mult(gb_a, 0)
            pltpu.sync_copy(fbuf, acc.at[db.at[0]], add=True)
        pltpu.async_copy(tab.at[sb.at[base]], gb_a, sem_ga)

        def body(i, carry):
            a = base + 2 * i
            b = a + 1
            nxt = jnp.minimum(a + 2, nch - 1)
            pltpu.make_async_copy(tab.at[sb.at[a]], gb_a, sem_ga).wait()
            pltpu.async_copy(tab.at[sb.at[b]], gb_b, sem_gb)
            mult(gb_a, a)
            pltpu.sync_copy(fbuf, acc.at[db.at[a]], add=True)
            pltpu.make_async_copy(tab.at[sb.at[b]], gb_b, sem_gb).wait()
            pltpu.async_copy(tab.at[sb.at[nxt]], gb_a, sem_ga)
            mult(gb_b, b)
            pltpu.sync_copy(fbuf, acc.at[db.at[b]], add=True)
            return carry
        lax.fori_loop(0, (nch - base) // 2, body, 0)
        pltpu.make_async_copy(tab.at[sb.at[0]], gb_a, sem_ga).wait()

    run_list(_C1, src1h.at[wid], dst1h.at[wid], val1h.at[wid], tab1)
    run_list(_C1, src2h.at[wid, 0], dst2h.at[wid, 0], val2h.at[wid, 0], tab2)
    run_list(_C1, src2h.at[wid, 1], dst2h.at[wid, 1], val2h.at[wid, 1], tab2)
    plsc.subcore_barrier()

    pltpu.sync_copy(acc.at[pl.ds(s * _RPT, _RPT)],
                    outh.at[c, pl.ds(s * _RPT, _RPT)])

    @pl.when(s == 0)
    def _():
        pltpu.sync_copy(acc.at[pl.ds(_NS * _RPT, _TAIL)],
                        outh.at[c, pl.ds(_NS * _RPT, _TAIL)])


def kernel(x, edge_index, adj_values, adj2_index, adj2_values, W1, W_out, b_out):
    f32 = jnp.float32

    rblk = 2000
    grid = (_N // rblk,)
    wa = W_out[:, 0:_HID]
    wbp = W_out[:, _HID:2 * _HID][_PERM, :]
    wcp = W_out[:, 2 * _HID:3 * _HID][_PERM, :]
    dense, t1, t2 = pl.pallas_call(
        _dense_body,
        grid=grid,
        in_specs=[
            pl.BlockSpec((rblk, _IN), lambda i: (i, 0)),
            pl.BlockSpec((_HID, _IN), lambda i: (0, 0)),
            pl.BlockSpec((_OUT, _HID), lambda i: (0, 0)),
            pl.BlockSpec((_OUT, _HID), lambda i: (0, 0)),
            pl.BlockSpec((_OUT, _HID), lambda i: (0, 0)),
            pl.BlockSpec((8, _OUT), lambda i: (0, 0)),
        ],
        out_specs=[
            pl.BlockSpec((rblk, _OUT), lambda i: (i, 0)),
            pl.BlockSpec((rblk, _OUT), lambda i: (i, 0)),
            pl.BlockSpec((rblk, _OUT), lambda i: (i, 0)),
        ],
        out_shape=[
            jax.ShapeDtypeStruct((_N, _OUT), f32),
            jax.ShapeDtypeStruct((_N, _OUT), jnp.bfloat16),
            jax.ShapeDtypeStruct((_N, _OUT), jnp.bfloat16),
        ],
    )(x, W1, wa, wbp, wcp, jnp.broadcast_to(b_out, (8, _OUT)))

    p1 = [(0, _E1P - _E1)]
    p2 = [(0, _E2P - _E2)]
    dst1 = jnp.pad(edge_index[0], p1).reshape(_NW, _C1, _K)
    src1 = jnp.pad(edge_index[1], p1).reshape(_NW, _C1, _K)
    val1 = jnp.pad(adj_values, p1).reshape(_NW, _C1, _K)
    dst2 = jnp.pad(adj2_index[0], p2).reshape(_NW, 2, _C1, _K)
    src2 = jnp.pad(adj2_index[1], p2).reshape(_NW, 2, _C1, _K)
    val2 = jnp.pad(adj2_values, p2).reshape(_NW, 2, _C1, _K)
    zeros = jnp.zeros((_N, _OUT), f32)

    mesh = plsc.VectorSubcoreMesh(core_axis_name="c", subcore_axis_name="s",
                                  num_cores=_NC, num_subcores=_NS)
    partial = pl.kernel(
        _sc_body,
        jax.ShapeDtypeStruct((_NC, _N, _OUT), f32),
        mesh=mesh,
        compiler_params=pltpu.CompilerParams(use_tc_tiling_on_sc=False,
                                             needs_layout_passes=False),
        scratch_types=[
            pltpu.VMEM_SHARED((_N, _OUT), f32),
            pltpu.VMEM_SHARED((_N, _OUT), jnp.bfloat16),
            pltpu.VMEM_SHARED((_N, _OUT), jnp.bfloat16),
            pltpu.VMEM((_C1, _K), jnp.int32),
            pltpu.VMEM((_C1, _K), jnp.int32),
            pltpu.VMEM((_C1, _K), f32),
            pltpu.VMEM((_K, _OUT), jnp.bfloat16),
            pltpu.VMEM((_K, _OUT), jnp.bfloat16),
            pltpu.VMEM((_K, _OUT), f32),
            pltpu.SemaphoreType.DMA,
            pltpu.SemaphoreType.DMA,
            pltpu.SemaphoreType.DMA,
        ],
    )(src1, dst1, val1, src2, dst2, val2, t1, t2, zeros)

    out = pl.pallas_call(
        _combine_body,
        grid=grid,
        in_specs=[
            pl.BlockSpec((rblk, _OUT), lambda i: (i, 0)),
            pl.BlockSpec((_NC, rblk, _OUT), lambda i: (0, i, 0)),
        ],
        out_specs=pl.BlockSpec((rblk, _OUT), lambda i: (i, 0)),
        out_shape=jax.ShapeDtypeStruct((_N, _OUT), f32),
    )(dense, partial)
    return out


# speedup vs baseline: 1.6095x; 1.1525x over previous
import functools

import jax
import jax.numpy as jnp
import numpy as np
from jax import lax
from jax.experimental import pallas as pl
from jax.experimental.pallas import tpu as pltpu
from jax.experimental.pallas import tpu_sc as plsc

_N = 10000
_IN = 128
_HID = 128
_OUT = 64
_E1 = 320000
_E2 = 640000

_NC = 2
_NS = 16
_NW = _NC * _NS
_K = 80
_C1 = -(-_E1 // (_NW * _K))
_C2 = -(-_E2 // (_NW * _K))
_E1P = _NW * _K * _C1
_E2P = _NW * _K * _C2
_RPT = 624
_TAIL = _N - _NS * _RPT

_PERM = np.empty(_OUT, np.int32)
for _g in range(0, _OUT, 32):
    for _t in range(16):
        _PERM[_g + 2 * _t] = _g + _t
        _PERM[_g + 2 * _t + 1] = _g + 16 + _t


def _dense_body(x_ref, w1_ref, wa_ref, wbp_ref, wcp_ref, b_ref,
                dense_ref, t1_ref, t2_ref):
    dims = (((1,), (1,)), ((), ()))
    h0 = lax.dot_general(x_ref[...], w1_ref[...], dims,
                         preferred_element_type=jnp.float32)
    dense_ref[...] = lax.dot_general(h0, wa_ref[...], dims,
                                     preferred_element_type=jnp.float32) + b_ref[0:1, :]
    t1_ref[...] = lax.dot_general(h0, wbp_ref[...], dims,
                                  preferred_element_type=jnp.float32).astype(jnp.bfloat16)
    t2_ref[...] = lax.dot_general(h0, wcp_ref[...], dims,
                                  preferred_element_type=jnp.float32).astype(jnp.bfloat16)


def _combine_body(dense_ref, p_ref, out_ref):
    out_ref[...] = dense_ref[...] + p_ref[0] + p_ref[1]


def _sc_body(src1h, dst1h, val1h, src2h, dst2h, val2h, t1h, t2h, zh, outh,
             acc, tab1, tab2, sb, db, vb, gb_a, gb_b, fb_a, fb_b,
             sem_ga, sem_gb, sem_sa, sem_sb, sem_t):
    c = lax.axis_index("c")
    s = lax.axis_index("s")
    wid = c * _NS + s

    t1cp = pltpu.make_async_copy(t1h.at[pl.ds(s * _RPT, _RPT)],
                                 tab1.at[pl.ds(s * _RPT, _RPT)], sem_t)
    t1cp.start()
    t2cp = pltpu.make_async_copy(t2h.at[pl.ds(s * _RPT, _RPT)],
                                 tab2.at[pl.ds(s * _RPT, _RPT)], sem_t)
    t2cp.start()

    @pl.when(s == 0)
    def _():
        pltpu.async_copy(t1h.at[pl.ds(_NS * _RPT, _TAIL)],
                         tab1.at[pl.ds(_NS * _RPT, _TAIL)], sem_t)
        pltpu.async_copy(t2h.at[pl.ds(_NS * _RPT, _TAIL)],
                         tab2.at[pl.ds(_NS * _RPT, _TAIL)], sem_t)

    pltpu.sync_copy(zh.at[pl.ds(s * _RPT, _RPT)], acc.at[pl.ds(s * _RPT, _RPT)])

    @pl.when(s == 0)
    def _():
        pltpu.sync_copy(zh.at[pl.ds(_NS * _RPT, _TAIL)],
                        acc.at[pl.ds(_NS * _RPT, _TAIL)])

    t1cp.wait()
    t2cp.wait()

    @pl.when(s == 0)
    def _():
        pltpu.make_async_copy(t1h.at[pl.ds(_NS * _RPT, _TAIL)],
                              tab1.at[pl.ds(_NS * _RPT, _TAIL)], sem_t).wait()
        pltpu.make_async_copy(t2h.at[pl.ds(_NS * _RPT, _TAIL)],
                              tab2.at[pl.ds(_NS * _RPT, _TAIL)], sem_t).wait()

    plsc.subcore_barrier()

    def mult(gb, fb, k):
        for g in range(_K // 16):
            vv = vb[k, pl.ds(g * 16, 16)]
            for jj in range(16):
                j = g * 16 + jj
                v = vv[jj]
                for h in range(_OUT // 32):
                    x32 = gb[j, pl.ds(h * 32, 32)]
                    lo, hi = plsc.unpack(x32, format=plsc.PackFormat.INTERLEAVED)
                    fb[j, pl.ds(h * 32, 16)] = lo * v
                    fb[j, pl.ds(h * 32 + 16, 16)] = hi * v

    def run_list(nch, srcsl, dstsl, valsl, tab):
        pltpu.sync_copy(srcsl, sb.at[pl.ds(0, nch)])
        pltpu.sync_copy(dstsl, db.at[pl.ds(0, nch)])
        pltpu.sync_copy(valsl, vb.at[pl.ds(0, nch)])

        base = nch % 2
        if base:
            pltpu.async_copy(tab.at[sb.at[0]], gb_a, sem_ga).wait()
            mult(gb_a, fb_a, 0)
            pltpu.sync_copy(fb_a, acc.at[db.at[0]], add=True)
        pltpu.async_copy(tab.at[sb.at[base]], gb_a, sem_ga)

        def body(i, carry):
            a = base + 2 * i
            b = a + 1
            nxt = jnp.minimum(a + 2, nch - 1)
            pltpu.make_async_copy(tab.at[sb.at[a]], gb_a, sem_ga).wait()

            @pl.when(i > 0)
            def _():
                pltpu.make_async_copy(fb_a, acc.at[db.at[a]], sem_sa).wait()

            pltpu.async_copy(tab.at[sb.at[b]], gb_b, sem_gb)
            mult(gb_a, fb_a, a)
            pltpu.async_copy(fb_a, acc.at[db.at[a]], sem_sa, add=True)
            pltpu.make_async_copy(tab.at[sb.at[b]], gb_b, sem_gb).wait()
            pltpu.async_copy(tab.at[sb.at[nxt]], gb_a, sem_ga)

            @pl.when(i > 0)
            def _():
                pltpu.make_async_copy(fb_b, acc.at[db.at[b]], sem_sb).wait()

            mult(gb_b, fb_b, b)
            pltpu.async_copy(fb_b, acc.at[db.at[b]], sem_sb, add=True)
            return carry
        lax.fori_loop(0, (nch - base) // 2, body, 0)
        pltpu.make_async_copy(tab.at[sb.at[0]], gb_a, sem_ga).wait()
        pltpu.make_async_copy(fb_a, acc.at[db.at[0]], sem_sa).wait()
        pltpu.make_async_copy(fb_b, acc.at[db.at[0]], sem_sb).wait()

    run_list(_C1, src1h.at[wid], dst1h.at[wid], val1h.at[wid], tab1)
    run_list(_C1, src2h.at[wid, 0], dst2h.at[wid, 0], val2h.at[wid, 0], tab2)
    run_list(_C1, src2h.at[wid, 1], dst2h.at[wid, 1], val2h.at[wid, 1], tab2)
    plsc.subcore_barrier()

    pltpu.sync_copy(acc.at[pl.ds(s * _RPT, _RPT)],
                    outh.at[c, pl.ds(s * _RPT, _RPT)])

    @pl.when(s == 0)
    def _():
        pltpu.sync_copy(acc.at[pl.ds(_NS * _RPT, _TAIL)],
                        outh.at[c, pl.ds(_NS * _RPT, _TAIL)])


def kernel(x, edge_index, adj_values, adj2_index, adj2_values, W1, W_out, b_out):
    f32 = jnp.float32

    rblk = 2000
    grid = (_N // rblk,)
    wa = W_out[:, 0:_HID]
    wbp = W_out[:, _HID:2 * _HID][_PERM, :]
    wcp = W_out[:, 2 * _HID:3 * _HID][_PERM, :]
    dense, t1, t2 = pl.pallas_call(
        _dense_body,
        grid=grid,
        in_specs=[
            pl.BlockSpec((rblk, _IN), lambda i: (i, 0)),
            pl.BlockSpec((_HID, _IN), lambda i: (0, 0)),
            pl.BlockSpec((_OUT, _HID), lambda i: (0, 0)),
            pl.BlockSpec((_OUT, _HID), lambda i: (0, 0)),
            pl.BlockSpec((_OUT, _HID), lambda i: (0, 0)),
            pl.BlockSpec((8, _OUT), lambda i: (0, 0)),
        ],
        out_specs=[
            pl.BlockSpec((rblk, _OUT), lambda i: (i, 0)),
            pl.BlockSpec((rblk, _OUT), lambda i: (i, 0)),
            pl.BlockSpec((rblk, _OUT), lambda i: (i, 0)),
        ],
        out_shape=[
            jax.ShapeDtypeStruct((_N, _OUT), f32),
            jax.ShapeDtypeStruct((_N, _OUT), jnp.bfloat16),
            jax.ShapeDtypeStruct((_N, _OUT), jnp.bfloat16),
        ],
    )(x, W1, wa, wbp, wcp, jnp.broadcast_to(b_out, (8, _OUT)))

    p1 = [(0, _E1P - _E1)]
    p2 = [(0, _E2P - _E2)]
    dst1 = jnp.pad(edge_index[0], p1).reshape(_NW, _C1, _K)
    src1 = jnp.pad(edge_index[1], p1).reshape(_NW, _C1, _K)
    val1 = jnp.pad(adj_values, p1).reshape(_NW, _C1, _K)
    dst2 = jnp.pad(adj2_index[0], p2).reshape(_NW, 2, _C1, _K)
    src2 = jnp.pad(adj2_index[1], p2).reshape(_NW, 2, _C1, _K)
    val2 = jnp.pad(adj2_values, p2).reshape(_NW, 2, _C1, _K)
    zeros = jnp.zeros((_N, _OUT), f32)

    mesh = plsc.VectorSubcoreMesh(core_axis_name="c", subcore_axis_name="s",
                                  num_cores=_NC, num_subcores=_NS)
    partial = pl.kernel(
        _sc_body,
        jax.ShapeDtypeStruct((_NC, _N, _OUT), f32),
        mesh=mesh,
        compiler_params=pltpu.CompilerParams(use_tc_tiling_on_sc=False,
                                             needs_layout_passes=False),
        scratch_types=[
            pltpu.VMEM_SHARED((_N, _OUT), f32),
            pltpu.VMEM_SHARED((_N, _OUT), jnp.bfloat16),
            pltpu.VMEM_SHARED((_N, _OUT), jnp.bfloat16),
            pltpu.VMEM((_C1, _K), jnp.int32),
            pltpu.VMEM((_C1, _K), jnp.int32),
            pltpu.VMEM((_C1, _K), f32),
            pltpu.VMEM((_K, _OUT), jnp.bfloat16),
            pltpu.VMEM((_K, _OUT), jnp.bfloat16),
            pltpu.VMEM((_K, _OUT), f32),
            pltpu.VMEM((_K, _OUT), f32),
            pltpu.SemaphoreType.DMA,
            pltpu.SemaphoreType.DMA,
            pltpu.SemaphoreType.DMA,
            pltpu.SemaphoreType.DMA,
            pltpu.SemaphoreType.DMA,
        ],
    )(src1, dst1, val1, src2, dst2, val2, t1, t2, zeros)

    out = pl.pallas_call(
        _combine_body,
        grid=grid,
        in_specs=[
            pl.BlockSpec((rblk, _OUT), lambda i: (i, 0)),
            pl.BlockSpec((_NC, rblk, _OUT), lambda i: (0, i, 0)),
        ],
        out_specs=pl.BlockSpec((rblk, _OUT), lambda i: (i, 0)),
        out_shape=jax.ShapeDtypeStruct((_N, _OUT), f32),
    )(dense, partial)
    return out
